# Initial kernel scaffold; baseline (speedup 1.0000x reference)
import jax
import jax.numpy as jnp
from jax.experimental import pallas as pl


def kernel(x, edge_index, fn_gamma, fn_beta, proj_w, proj_b, blk_w1, blk_b1, blk_w2, blk_b2, blk_gamma, blk_beta, pred_w, pred_b):
    raise NotImplementedError("write your pallas kernel here")


# speedup vs baseline: 3.0551x; 3.0551x over previous
import jax
import jax.numpy as jnp
from jax import lax
from jax.experimental import pallas as pl
from jax.experimental.pallas import tpu as pltpu
from jax.experimental.pallas import tpu_sc as plsc

N, E, D, H, OUT = 10000, 320000, 128, 128, 128
NUM_BLOCKS = 3
NC, NS = 2, 16
NW = NC * NS
CHUNK = 128
CPW = 80
HALF = 40
EPW = CHUNK * CPW
E_PAD = NW * EPW
ZROWS = 632
ACC_R = NS * ZROWS
EPS = 1e-5


def _agg_body(h_hbm, src_hbm, dst_hbm, zeros_hbm, out_hbm,
              acc, src_v, dst_v, rows0, rows1, sem0, sem1):
    c = lax.axis_index("c")
    s = lax.axis_index("s")
    wid = s * NC + c
    pltpu.sync_copy(zeros_hbm, acc.at[pl.ds(s * ZROWS, ZROWS)])
    plsc.subcore_barrier()

    for half in range(CPW // HALF):
        base = half * HALF
        pltpu.sync_copy(src_hbm.at[wid, pl.ds(base, HALF)], src_v)
        pltpu.sync_copy(dst_hbm.at[wid, pl.ds(base, HALF)], dst_v)
        pltpu.async_copy(h_hbm.at[src_v.at[0]], rows0, sem0)
        pltpu.async_copy(h_hbm.at[src_v.at[1]], rows1, sem1)

        def step(i, carry):
            j0 = 2 * i
            pltpu.make_async_copy(h_hbm.at[src_v.at[j0]], rows0, sem0).wait()
            pltpu.sync_copy(rows0, acc.at[dst_v.at[j0]], add=True)
            pltpu.async_copy(h_hbm.at[src_v.at[j0 + 2]], rows0, sem0)
            pltpu.make_async_copy(h_hbm.at[src_v.at[j0 + 1]], rows1, sem1).wait()
            pltpu.sync_copy(rows1, acc.at[dst_v.at[j0 + 1]], add=True)
            pltpu.async_copy(h_hbm.at[src_v.at[j0 + 3]], rows1, sem1)
            return carry

        lax.fori_loop(0, HALF // 2 - 1, step, 0)
        pltpu.make_async_copy(h_hbm.at[src_v.at[HALF - 2]], rows0, sem0).wait()
        pltpu.sync_copy(rows0, acc.at[dst_v.at[HALF - 2]], add=True)
        pltpu.make_async_copy(h_hbm.at[src_v.at[HALF - 1]], rows1, sem1).wait()
        pltpu.sync_copy(rows1, acc.at[dst_v.at[HALF - 1]], add=True)
    plsc.subcore_barrier()
    pltpu.sync_copy(acc.at[pl.ds(s * ZROWS, ZROWS)],
                    out_hbm.at[c, pl.ds(s * ZROWS, ZROWS)])


_SC_AGG_CACHE = []


def _sc_agg_kernel():
    if not _SC_AGG_CACHE:
        _SC_AGG_CACHE.append(pl.kernel(
            _agg_body,
            out_type=jax.ShapeDtypeStruct((NC, ACC_R, H), jnp.float32),
            mesh=plsc.VectorSubcoreMesh(core_axis_name="c",
                                        subcore_axis_name="s",
                                        num_cores=NC, num_subcores=NS),
            scratch_types=[
                pltpu.VMEM_SHARED((ACC_R, H), jnp.float32),
                pltpu.VMEM((HALF, CHUNK), jnp.int32),
                pltpu.VMEM((HALF, CHUNK), jnp.int32),
                pltpu.VMEM((CHUNK, H), jnp.float32),
                pltpu.VMEM((CHUNK, H), jnp.float32),
                pltpu.SemaphoreType.DMA,
                pltpu.SemaphoreType.DMA,
            ],
        ))
    return _SC_AGG_CACHE[0]


def _tc_pre_body(x_ref, g_ref, b_ref, w_ref, wb_ref, out_ref):
    x = x_ref[...]
    mu = jnp.mean(x, axis=0, keepdims=True)
    xc = x - mu
    var = jnp.mean(xc * xc, axis=0, keepdims=True)
    xn = xc * lax.rsqrt(var + EPS) * g_ref[...] + b_ref[...]
    h = jnp.dot(xn, w_ref[...], preferred_element_type=jnp.float32) + wb_ref[...]
    out_ref[...] = jnp.maximum(h, 0.0)


def _tc_block_body(h_ref, p_ref, w1_ref, b1_ref, w2_ref, b2_ref, g_ref, be_ref,
                   out_ref):
    h = h_ref[...]
    z = h + p_ref[0, :N, :] + p_ref[1, :N, :]
    z = jnp.maximum(jnp.dot(z, w1_ref[...], preferred_element_type=jnp.float32)
                    + b1_ref[...], 0.0)
    z = jnp.dot(z, w2_ref[...], preferred_element_type=jnp.float32) + b2_ref[...]
    mu = jnp.mean(z, axis=0, keepdims=True)
    zc = z - mu
    var = jnp.mean(zc * zc, axis=0, keepdims=True)
    z = zc * lax.rsqrt(var + EPS) * g_ref[...] + be_ref[...]
    out_ref[...] = jnp.maximum(z, 0.0) + h


def _tc_post_body(h_ref, w_ref, b_ref, emb_ref, log_ref):
    pooled = jnp.mean(h_ref[...], axis=0, keepdims=True)
    pooled8 = jnp.broadcast_to(pooled, (8, H))
    emb_ref[...] = pooled8
    log_ref[...] = (jnp.dot(pooled8, w_ref[...],
                            preferred_element_type=jnp.float32) + b_ref[...])


def kernel(x, edge_index, fn_gamma, fn_beta, proj_w, proj_b, blk_w1, blk_b1,
           blk_w2, blk_b2, blk_gamma, blk_beta, pred_w, pred_b):
    f32 = jnp.float32
    src = edge_index[0]
    dst = edge_index[1]
    pad = E_PAD - E
    src_p = jnp.concatenate([src, jnp.zeros((pad,), jnp.int32)])
    dst_p = jnp.concatenate([dst, jnp.full((pad,), N, jnp.int32)])
    src_p = src_p.reshape(NW, CPW, CHUNK)
    dst_p = dst_p.reshape(NW, CPW, CHUNK)
    zeros = jnp.zeros((ZROWS, H), f32)

    h = pl.pallas_call(
        _tc_pre_body,
        out_shape=jax.ShapeDtypeStruct((N, H), f32),
    )(x, fn_gamma.reshape(1, D), fn_beta.reshape(1, D), proj_w,
      proj_b.reshape(1, H))

    for i in range(NUM_BLOCKS):
        parts = _sc_agg_kernel()(h, src_p, dst_p, zeros)
        h = pl.pallas_call(
            _tc_block_body,
            out_shape=jax.ShapeDtypeStruct((N, H), f32),
        )(h, parts, blk_w1[i], blk_b1[i].reshape(1, H), blk_w2[i],
          blk_b2[i].reshape(1, H), blk_gamma[i].reshape(1, H),
          blk_beta[i].reshape(1, H))

    emb8, log8 = pl.pallas_call(
        _tc_post_body,
        out_shape=(jax.ShapeDtypeStruct((8, H), f32),
                   jax.ShapeDtypeStruct((8, OUT), f32)),
    )(h, pred_w, pred_b.reshape(1, OUT))
    return emb8[:1], log8[:1]


# speedup vs baseline: 3.4562x; 1.1313x over previous
import jax
import jax.numpy as jnp
from jax import lax
from jax.experimental import pallas as pl
from jax.experimental.pallas import tpu as pltpu
from jax.experimental.pallas import tpu_sc as plsc

N, E, D, H, OUT = 10000, 320000, 128, 128, 128
NUM_BLOCKS = 3
NC, NS = 2, 16
NW = NC * NS
CHUNK = 128
HALF = 32
FAST_C = 0
FAST_HPW = 4
SLOW_HPW = 1
TOT_CHUNKS = NS * HALF * (FAST_HPW + SLOW_HPW)
FAST_CHUNKS = NS * HALF * FAST_HPW
E_PAD = TOT_CHUNKS * CHUNK
ZROWS = 632
ACC_R = NS * ZROWS
EPS = 1e-5


def _agg_body(h_hbm, src_hbm, dst_hbm, zeros_hbm, out_hbm,
              acc, src_v, dst_v, rows0, rows1, sem0, sem1):
    c = lax.axis_index("c")
    s = lax.axis_index("s")
    pltpu.sync_copy(zeros_hbm, acc.at[pl.ds(s * ZROWS, ZROWS)])
    plsc.subcore_barrier()

    is_fast = c == FAST_C
    n_refills = jnp.where(is_fast, FAST_HPW, SLOW_HPW)
    base = jnp.where(is_fast, s * (FAST_HPW * HALF),
                     FAST_CHUNKS + s * (SLOW_HPW * HALF))

    def refill(half, carry):
        cb = pl.multiple_of(base + half * HALF, 8)
        pltpu.sync_copy(src_hbm.at[pl.ds(cb, HALF)], src_v)
        pltpu.sync_copy(dst_hbm.at[pl.ds(cb, HALF)], dst_v)
        pltpu.async_copy(h_hbm.at[src_v.at[0]], rows0, sem0)
        pltpu.async_copy(h_hbm.at[src_v.at[1]], rows1, sem1)

        def step(i, carry2):
            j0 = 2 * i
            pltpu.make_async_copy(h_hbm.at[src_v.at[j0]], rows0, sem0).wait()
            pltpu.sync_copy(rows0, acc.at[dst_v.at[j0]], add=True)
            pltpu.async_copy(h_hbm.at[src_v.at[j0 + 2]], rows0, sem0)
            pltpu.make_async_copy(h_hbm.at[src_v.at[j0 + 1]], rows1, sem1).wait()
            pltpu.sync_copy(rows1, acc.at[dst_v.at[j0 + 1]], add=True)
            pltpu.async_copy(h_hbm.at[src_v.at[j0 + 3]], rows1, sem1)
            return carry2

        lax.fori_loop(0, HALF // 2 - 1, step, 0)
        pltpu.make_async_copy(h_hbm.at[src_v.at[HALF - 2]], rows0, sem0).wait()
        pltpu.sync_copy(rows0, acc.at[dst_v.at[HALF - 2]], add=True)
        pltpu.make_async_copy(h_hbm.at[src_v.at[HALF - 1]], rows1, sem1).wait()
        pltpu.sync_copy(rows1, acc.at[dst_v.at[HALF - 1]], add=True)
        return carry

    lax.fori_loop(0, n_refills, refill, 0)
    plsc.subcore_barrier()
    pltpu.sync_copy(acc.at[pl.ds(s * ZROWS, ZROWS)],
                    out_hbm.at[c, pl.ds(s * ZROWS, ZROWS)])


_SC_AGG_CACHE = []


def _sc_agg_kernel():
    if not _SC_AGG_CACHE:
        _SC_AGG_CACHE.append(pl.kernel(
            _agg_body,
            out_type=jax.ShapeDtypeStruct((NC, ACC_R, H), jnp.float32),
            mesh=plsc.VectorSubcoreMesh(core_axis_name="c",
                                        subcore_axis_name="s",
                                        num_cores=NC, num_subcores=NS),
            scratch_types=[
                pltpu.VMEM_SHARED((ACC_R, H), jnp.float32),
                pltpu.VMEM((HALF, CHUNK), jnp.int32),
                pltpu.VMEM((HALF, CHUNK), jnp.int32),
                pltpu.VMEM((CHUNK, H), jnp.float32),
                pltpu.VMEM((CHUNK, H), jnp.float32),
                pltpu.SemaphoreType.DMA,
                pltpu.SemaphoreType.DMA,
            ],
        ))
    return _SC_AGG_CACHE[0]


def _tc_pre_body(x_ref, g_ref, b_ref, w_ref, wb_ref, out_ref):
    x = x_ref[...]
    mu = jnp.mean(x, axis=0, keepdims=True)
    xc = x - mu
    var = jnp.mean(xc * xc, axis=0, keepdims=True)
    xn = xc * lax.rsqrt(var + EPS) * g_ref[...] + b_ref[...]
    h = jnp.dot(xn, w_ref[...], preferred_element_type=jnp.float32) + wb_ref[...]
    out_ref[...] = jnp.maximum(h, 0.0)


def _tc_block_body(h_ref, p_ref, w1_ref, b1_ref, w2_ref, b2_ref, g_ref, be_ref,
                   out_ref):
    h = h_ref[...]
    z = h + p_ref[0, :N, :] + p_ref[1, :N, :]
    z = jnp.maximum(jnp.dot(z, w1_ref[...], preferred_element_type=jnp.float32)
                    + b1_ref[...], 0.0)
    z = jnp.dot(z, w2_ref[...], preferred_element_type=jnp.float32) + b2_ref[...]
    mu = jnp.mean(z, axis=0, keepdims=True)
    zc = z - mu
    var = jnp.mean(zc * zc, axis=0, keepdims=True)
    z = zc * lax.rsqrt(var + EPS) * g_ref[...] + be_ref[...]
    out_ref[...] = jnp.maximum(z, 0.0) + h


def _tc_post_body(h_ref, w_ref, b_ref, emb_ref, log_ref):
    pooled = jnp.mean(h_ref[...], axis=0, keepdims=True)
    pooled8 = jnp.broadcast_to(pooled, (8, H))
    emb_ref[...] = pooled8
    log_ref[...] = (jnp.dot(pooled8, w_ref[...],
                            preferred_element_type=jnp.float32) + b_ref[...])


def kernel(x, edge_index, fn_gamma, fn_beta, proj_w, proj_b, blk_w1, blk_b1,
           blk_w2, blk_b2, blk_gamma, blk_beta, pred_w, pred_b):
    f32 = jnp.float32
    src = edge_index[0]
    dst = edge_index[1]
    pad = E_PAD - E
    src_p = jnp.concatenate([src, jnp.zeros((pad,), jnp.int32)])
    dst_p = jnp.concatenate([dst, jnp.full((pad,), N, jnp.int32)])
    src_p = src_p.reshape(TOT_CHUNKS, CHUNK)
    dst_p = dst_p.reshape(TOT_CHUNKS, CHUNK)
    zeros = jnp.zeros((ZROWS, H), f32)

    h = pl.pallas_call(
        _tc_pre_body,
        out_shape=jax.ShapeDtypeStruct((N, H), f32),
    )(x, fn_gamma.reshape(1, D), fn_beta.reshape(1, D), proj_w,
      proj_b.reshape(1, H))

    for i in range(NUM_BLOCKS):
        parts = _sc_agg_kernel()(h, src_p, dst_p, zeros)
        h = pl.pallas_call(
            _tc_block_body,
            out_shape=jax.ShapeDtypeStruct((N, H), f32),
        )(h, parts, blk_w1[i], blk_b1[i].reshape(1, H), blk_w2[i],
          blk_b2[i].reshape(1, H), blk_gamma[i].reshape(1, H),
          blk_beta[i].reshape(1, H))

    emb8, log8 = pl.pallas_call(
        _tc_post_body,
        out_shape=(jax.ShapeDtypeStruct((8, H), f32),
                   jax.ShapeDtypeStruct((8, OUT), f32)),
    )(h, pred_w, pred_b.reshape(1, OUT))
    return emb8[:1], log8[:1]
